# Initial kernel scaffold; baseline (speedup 1.0000x reference)
import functools

import jax
import jax.numpy as jnp
from jax import lax
from jax.experimental import pallas as pl
from jax.experimental.pallas import tpu as pltpu
from jax.experimental.pallas import tpu_sc as plsc

N = 10000
E = 320000
H = 128
NUM_GRAPHS = 64

NUM_CORES = 2
NUM_SUB = 16
NTILE = NUM_CORES * NUM_SUB
CHUNK = 128
CPT = 80
EPT = CPT * CHUNK
E_PAD = NTILE * EPT
N_PAD = 10240
RPS = N_PAD // NUM_SUB


def _edge_body(h_hbm, src_hbm, dst_hbm, ea_hbm, we_hbm, be_hbm, zero_hbm,
               out_hbm, src_v, dst_v, ea_v, we_v, be_v, rows_v, aggr_sh, sem):
    c = lax.axis_index("c")
    s = lax.axis_index("s")
    t = c * NUM_SUB + s

    pltpu.sync_copy(src_hbm.at[t], src_v)
    pltpu.sync_copy(dst_hbm.at[t], dst_v)
    pltpu.sync_copy(ea_hbm.at[t], ea_v)
    pltpu.sync_copy(we_hbm, we_v)
    pltpu.sync_copy(be_hbm, be_v)

    pltpu.sync_copy(zero_hbm, rows_v)
    for q in range(RPS // CHUNK):
        pltpu.sync_copy(rows_v, aggr_sh.at[pl.ds(s * RPS + q * CHUNK, CHUNK)])
    plsc.subcore_barrier()

    W = [[we_v[k, pl.ds(v * 16, 16)] for v in range(8)] for k in range(3)]
    B = [be_v[pl.ds(v * 16, 16)] for v in range(8)]

    def compute_chunk(j):
        def g_body(g, _):
            gi = j * 8 + g
            ea0 = ea_v[0, gi]
            ea1 = ea_v[1, gi]
            ea2 = ea_v[2, gi]

            def l_body(l, _2):
                i = g * 16 + l
                lidx = jnp.full((16,), l, dtype=jnp.int32)
                s0 = jnp.take(ea0, lidx, mode="promise_in_bounds")
                s1 = jnp.take(ea1, lidx, mode="promise_in_bounds")
                s2 = jnp.take(ea2, lidx, mode="promise_in_bounds")
                for v in range(8):
                    sl = pl.ds(v * 16, 16)
                    hv = rows_v[i, sl]
                    m = hv + s0 * W[0][v] + s1 * W[1][v] + s2 * W[2][v] + B[v]
                    rows_v[i, sl] = jnp.maximum(m, 0.0)
                return 0

            lax.fori_loop(0, 16, l_body, 0)
            return 0

        lax.fori_loop(0, 8, g_body, 0)

    def main_body(j, _):
        pltpu.async_copy(h_hbm.at[src_v.at[j]], rows_v, sem).wait()
        compute_chunk(j)
        pltpu.sync_copy(rows_v, aggr_sh.at[dst_v.at[j]], add=True)
        return 0

    lax.fori_loop(0, CPT, main_body, 0)
    plsc.subcore_barrier()

    pltpu.sync_copy(aggr_sh.at[pl.ds(s * RPS, RPS)],
                    out_hbm.at[c, pl.ds(s * RPS, RPS)])


_edge_kernel = functools.partial(
    pl.kernel,
    out_type=jax.ShapeDtypeStruct((NUM_CORES, N_PAD, H), jnp.float32),
    mesh=plsc.VectorSubcoreMesh(core_axis_name="c", subcore_axis_name="s"),
    scratch_types=[
        pltpu.VMEM((CPT, CHUNK), jnp.int32),
        pltpu.VMEM((CPT, CHUNK), jnp.int32),
        pltpu.VMEM((3, CPT * 8, 16), jnp.float32),
        pltpu.VMEM((3, H), jnp.float32),
        pltpu.VMEM((H,), jnp.float32),
        pltpu.VMEM((CHUNK, H), jnp.float32),
        pltpu.VMEM_SHARED((N_PAD, H), jnp.float32),
        pltpu.SemaphoreType.DMA,
    ],
)(_edge_body)


def _layer_body(h_ref, agg_ref, wa_ref, ba_ref, wb_ref, bb_ref, gm_ref,
                bt_ref, o_ref):
    z = h_ref[...] + agg_ref[0, :N, :] + agg_ref[1, :N, :]
    u = jnp.dot(z, wa_ref[...], preferred_element_type=jnp.float32)
    u = jnp.maximum(u + ba_ref[...], 0.0)
    y = jnp.dot(u, wb_ref[...], preferred_element_type=jnp.float32)
    y = y + bb_ref[...]
    mu = jnp.mean(y, axis=0, keepdims=True)
    yc = y - mu
    var = jnp.mean(yc * yc, axis=0, keepdims=True)
    o_ref[...] = jnp.maximum(
        gm_ref[...] * yc * lax.rsqrt(var + 1e-5) + bt_ref[...], 0.0)


_layer_tc = pl.pallas_call(
    _layer_body,
    out_shape=jax.ShapeDtypeStruct((N, H), jnp.float32),
)


def _final_body(h_ref, agg_ref, wa_ref, ba_ref, wb_ref, bb_ref, gm_ref,
                bt_ref, batch_ref, wf1_ref, bf1_ref, wf2_ref, bf2_ref, o_ref):
    z = h_ref[...] + agg_ref[0, :N, :] + agg_ref[1, :N, :]
    u = jnp.dot(z, wa_ref[...], preferred_element_type=jnp.float32)
    u = jnp.maximum(u + ba_ref[...], 0.0)
    y = jnp.dot(u, wb_ref[...], preferred_element_type=jnp.float32)
    y = y + bb_ref[...]
    mu = jnp.mean(y, axis=0, keepdims=True)
    yc = y - mu
    var = jnp.mean(yc * yc, axis=0, keepdims=True)
    h3 = jnp.maximum(
        gm_ref[...] * yc * lax.rsqrt(var + 1e-5) + bt_ref[...], 0.0)
    seg = lax.broadcasted_iota(jnp.int32, (NUM_GRAPHS, N), 0)
    onehot = (seg == jnp.broadcast_to(batch_ref[...], (NUM_GRAPHS, N))
              ).astype(jnp.float32)
    sums = jnp.dot(onehot, h3, preferred_element_type=jnp.float32)
    cnts = jnp.sum(onehot, axis=1, keepdims=True)
    pooled = sums / jnp.maximum(cnts, 1.0)
    t1 = jnp.dot(pooled, wf1_ref[...], preferred_element_type=jnp.float32)
    t1 = jnp.maximum(t1 + bf1_ref[...], 0.0)
    o_ref[...] = (jnp.dot(t1, wf2_ref[...], preferred_element_type=jnp.float32)
                  + bf2_ref[...])


_final_tc = pl.pallas_call(
    _final_body,
    out_shape=jax.ShapeDtypeStruct((NUM_GRAPHS, 1), jnp.float32),
)


def kernel(x, edge_index, edge_attr, batch,
           We1, be1, Wa1, ba1, Wb1, bb1, gamma1, beta1,
           We2, be2, Wa2, ba2, Wb2, bb2, gamma2, beta2,
           We3, be3, Wa3, ba3, Wb3, bb3, gamma3, beta3,
           Wf1, bf1, Wf2, bf2):
    pad = E_PAD - E
    src_p = jnp.concatenate(
        [edge_index[0], jnp.zeros((pad,), jnp.int32)]).reshape(
            NTILE, CPT, CHUNK)
    dummy = N + (jnp.arange(pad, dtype=jnp.int32) % (N_PAD - N))
    dst_p = jnp.concatenate([edge_index[1], dummy]).reshape(NTILE, CPT, CHUNK)
    ea_p = jnp.pad(edge_attr.T, ((0, 0), (0, pad))).reshape(
        3, NTILE, CPT * 8, 16).transpose(1, 0, 2, 3)
    zeros128 = jnp.zeros((CHUNK, H), jnp.float32)

    def edge(h, We, be):
        return _edge_kernel(h, src_p, dst_p, ea_p, We, be, zeros128)

    r1 = lambda a: a.reshape(1, -1)
    agg1 = edge(x, We1, be1)
    h1 = _layer_tc(x, agg1, Wa1, r1(ba1), Wb1, r1(bb1), r1(gamma1), r1(beta1))
    agg2 = edge(h1, We2, be2)
    h2 = _layer_tc(h1, agg2, Wa2, r1(ba2), Wb2, r1(bb2), r1(gamma2), r1(beta2))
    agg3 = edge(h2, We3, be3)
    out = _final_tc(h2, agg3, Wa3, r1(ba3), Wb3, r1(bb3), r1(gamma3),
                    r1(beta3), batch.reshape(1, N), Wf1, r1(bf1),
                    Wf2, r1(bf2))
    return out


# speedup vs baseline: 2.4521x; 2.4521x over previous
import functools

import jax
import jax.numpy as jnp
from jax import lax
from jax.experimental import pallas as pl
from jax.experimental.pallas import tpu as pltpu
from jax.experimental.pallas import tpu_sc as plsc

N = 10000
E = 320000
H = 128
NUM_GRAPHS = 64

NUM_CORES = 2
NUM_SUB = 16
NTILE = NUM_CORES * NUM_SUB
CHUNK = 128
CPT = 80
EPT = CPT * CHUNK
E_PAD = NTILE * EPT
N_PAD = 10240
RPS = N_PAD // NUM_SUB
SUP = 16
NSUP = CPT // SUP


def _splat(vec, l):
    idx = jnp.full((16, 1), l, dtype=jnp.int32)
    dnums = lax.GatherDimensionNumbers(
        offset_dims=(), collapsed_slice_dims=(0,), start_index_map=(0,))
    return lax.gather(vec, idx, dnums, (1,),
                      mode=lax.GatherScatterMode.PROMISE_IN_BOUNDS)


def _edge_body(h_hbm, src_hbm, dst_hbm, ea_hbm, we_hbm, be_hbm, zero_hbm,
               out_hbm, src_v, dst_v, ea_v, we_v, be_v, rows_v, aggr_sh, sem):
    c = lax.axis_index("c")
    s = lax.axis_index("s")
    t = c * NUM_SUB + s

    pltpu.sync_copy(we_hbm, we_v)
    pltpu.sync_copy(be_hbm, be_v)

    pltpu.sync_copy(zero_hbm, rows_v)
    for q in range(RPS // CHUNK):
        pltpu.sync_copy(rows_v, aggr_sh.at[pl.ds(s * RPS + q * CHUNK, CHUNK)])
    plsc.subcore_barrier()

    W = [[we_v[k, pl.ds(v * 16, 16)] for v in range(8)] for k in range(3)]
    B = [be_v[pl.ds(v * 16, 16)] for v in range(8)]

    def compute_chunk(jj):
        for g in range(8):
            ea0 = ea_v[0, jj, pl.ds(g * 16, 16)]
            ea1 = ea_v[1, jj, pl.ds(g * 16, 16)]
            ea2 = ea_v[2, jj, pl.ds(g * 16, 16)]

            def l_body(l, _2, g=g, ea0=ea0, ea1=ea1, ea2=ea2):
                i = g * 16 + l
                s0 = _splat(ea0, l)
                s1 = _splat(ea1, l)
                s2 = _splat(ea2, l)
                for v in range(8):
                    sl = pl.ds(v * 16, 16)
                    hv = rows_v[i, sl]
                    m = hv + s0 * W[0][v] + s1 * W[1][v] + s2 * W[2][v] + B[v]
                    rows_v[i, sl] = jnp.maximum(m, 0.0)
                return 0

            lax.fori_loop(0, 16, l_body, 0)

    def sup_body(u, _):
        pltpu.sync_copy(src_hbm.at[t, pl.ds(u * SUP, SUP)], src_v)
        pltpu.sync_copy(dst_hbm.at[t, pl.ds(u * SUP, SUP)], dst_v)
        pltpu.sync_copy(ea_hbm.at[t, :, pl.ds(u * SUP, SUP)], ea_v)

        def chunk_body(jj, _2):
            pltpu.async_copy(h_hbm.at[src_v.at[jj]], rows_v, sem).wait()
            compute_chunk(jj)
            pltpu.sync_copy(rows_v, aggr_sh.at[dst_v.at[jj]], add=True)
            return 0

        lax.fori_loop(0, SUP, chunk_body, 0)
        return 0

    lax.fori_loop(0, NSUP, sup_body, 0)
    plsc.subcore_barrier()

    pltpu.sync_copy(aggr_sh.at[pl.ds(s * RPS, RPS)],
                    out_hbm.at[c, pl.ds(s * RPS, RPS)])


@functools.cache
def _get_edge_kernel():
  return functools.partial(
    pl.kernel,
    out_type=jax.ShapeDtypeStruct((NUM_CORES, N_PAD, H), jnp.float32),
    mesh=plsc.VectorSubcoreMesh(core_axis_name="c", subcore_axis_name="s",
                                num_cores=NUM_CORES, num_subcores=NUM_SUB),
    scratch_types=[
        pltpu.VMEM((SUP, CHUNK), jnp.int32),
        pltpu.VMEM((SUP, CHUNK), jnp.int32),
        pltpu.VMEM((3, SUP, CHUNK), jnp.float32),
        pltpu.VMEM((3, H), jnp.float32),
        pltpu.VMEM((H,), jnp.float32),
        pltpu.VMEM((CHUNK, H), jnp.float32),
        pltpu.VMEM_SHARED((N_PAD, H), jnp.float32),
        pltpu.SemaphoreType.DMA,
    ],
  )(_edge_body)


def _layer_body(h_ref, agg_ref, wa_ref, ba_ref, wb_ref, bb_ref, gm_ref,
                bt_ref, o_ref):
    z = h_ref[...] + agg_ref[0, :N, :] + agg_ref[1, :N, :]
    u = jnp.dot(z, wa_ref[...], preferred_element_type=jnp.float32)
    u = jnp.maximum(u + ba_ref[...], 0.0)
    y = jnp.dot(u, wb_ref[...], preferred_element_type=jnp.float32)
    y = y + bb_ref[...]
    mu = jnp.mean(y, axis=0, keepdims=True)
    yc = y - mu
    var = jnp.mean(yc * yc, axis=0, keepdims=True)
    o_ref[...] = jnp.maximum(
        gm_ref[...] * yc * lax.rsqrt(var + 1e-5) + bt_ref[...], 0.0)


_layer_tc = pl.pallas_call(
    _layer_body,
    out_shape=jax.ShapeDtypeStruct((N, H), jnp.float32),
)


def _final_body(h_ref, agg_ref, wa_ref, ba_ref, wb_ref, bb_ref, gm_ref,
                bt_ref, batch_ref, wf1_ref, bf1_ref, wf2_ref, bf2_ref, o_ref):
    z = h_ref[...] + agg_ref[0, :N, :] + agg_ref[1, :N, :]
    u = jnp.dot(z, wa_ref[...], preferred_element_type=jnp.float32)
    u = jnp.maximum(u + ba_ref[...], 0.0)
    y = jnp.dot(u, wb_ref[...], preferred_element_type=jnp.float32)
    y = y + bb_ref[...]
    mu = jnp.mean(y, axis=0, keepdims=True)
    yc = y - mu
    var = jnp.mean(yc * yc, axis=0, keepdims=True)
    h3 = jnp.maximum(
        gm_ref[...] * yc * lax.rsqrt(var + 1e-5) + bt_ref[...], 0.0)
    seg = lax.broadcasted_iota(jnp.int32, (NUM_GRAPHS, N), 0)
    onehot = (seg == jnp.broadcast_to(batch_ref[...], (NUM_GRAPHS, N))
              ).astype(jnp.float32)
    sums = jnp.dot(onehot, h3, preferred_element_type=jnp.float32)
    cnts = jnp.sum(onehot, axis=1, keepdims=True)
    pooled = sums / jnp.maximum(cnts, 1.0)
    t1 = jnp.dot(pooled, wf1_ref[...], preferred_element_type=jnp.float32)
    t1 = jnp.maximum(t1 + bf1_ref[...], 0.0)
    o_ref[...] = (jnp.dot(t1, wf2_ref[...], preferred_element_type=jnp.float32)
                  + bf2_ref[...])


_final_tc = pl.pallas_call(
    _final_body,
    out_shape=jax.ShapeDtypeStruct((NUM_GRAPHS, 1), jnp.float32),
)


def kernel(x, edge_index, edge_attr, batch,
           We1, be1, Wa1, ba1, Wb1, bb1, gamma1, beta1,
           We2, be2, Wa2, ba2, Wb2, bb2, gamma2, beta2,
           We3, be3, Wa3, ba3, Wb3, bb3, gamma3, beta3,
           Wf1, bf1, Wf2, bf2):
    pad = E_PAD - E
    src_p = jnp.concatenate(
        [edge_index[0], jnp.zeros((pad,), jnp.int32)]).reshape(
            NTILE, CPT, CHUNK)
    dummy = N + (jnp.arange(pad, dtype=jnp.int32) % (N_PAD - N))
    dst_p = jnp.concatenate([edge_index[1], dummy]).reshape(NTILE, CPT, CHUNK)
    ea_p = jnp.pad(edge_attr.T, ((0, 0), (0, pad))).reshape(
        3, NTILE, CPT, CHUNK).transpose(1, 0, 2, 3)
    zeros128 = jnp.zeros((CHUNK, H), jnp.float32)

    def edge(h, We, be):
        return _get_edge_kernel()(h, src_p, dst_p, ea_p, We, be, zeros128)

    r1 = lambda a: a.reshape(1, -1)
    agg1 = edge(x, We1, be1)
    h1 = _layer_tc(x, agg1, Wa1, r1(ba1), Wb1, r1(bb1), r1(gamma1), r1(beta1))
    agg2 = edge(h1, We2, be2)
    h2 = _layer_tc(h1, agg2, Wa2, r1(ba2), Wb2, r1(bb2), r1(gamma2), r1(beta2))
    agg3 = edge(h2, We3, be3)
    out = _final_tc(h2, agg3, Wa3, r1(ba3), Wb3, r1(bb3), r1(gamma3),
                    r1(beta3), batch.reshape(1, N), Wf1, r1(bf1),
                    Wf2, r1(bf2))
    return out


# speedup vs baseline: 2.8693x; 1.1701x over previous
import functools

import jax
import jax.numpy as jnp
from jax import lax
from jax.experimental import pallas as pl
from jax.experimental.pallas import tpu as pltpu
from jax.experimental.pallas import tpu_sc as plsc

N = 10000
E = 320000
H = 128
NUM_GRAPHS = 64

NUM_CORES = 2
NUM_SUB = 16
NTILE = NUM_CORES * NUM_SUB
CHUNK = 128
CPT = 80
EPT = CPT * CHUNK
E_PAD = NTILE * EPT
N_PAD = 10240
RPS = N_PAD // NUM_SUB
SUP = 8
NSUP = CPT // SUP


def _splat(vec, l):
    idx = jnp.full((16, 1), l, dtype=jnp.int32)
    dnums = lax.GatherDimensionNumbers(
        offset_dims=(), collapsed_slice_dims=(0,), start_index_map=(0,))
    return lax.gather(vec, idx, dnums, (1,),
                      mode=lax.GatherScatterMode.PROMISE_IN_BOUNDS)


def _edge_body(h_hbm, src_hbm, dst_hbm, ea_hbm, we_hbm, be_hbm, zero_hbm,
               out_hbm, src_v, dst_v, ea_v, we_v, be_v, rows_a, rows_b,
               aggr_sh, gsem_a, gsem_b, ssem_a, ssem_b):
    c = lax.axis_index("c")
    s = lax.axis_index("s")
    t = c * NUM_SUB + s

    pltpu.sync_copy(we_hbm, we_v)
    pltpu.sync_copy(be_hbm, be_v)

    pltpu.sync_copy(zero_hbm, rows_a)
    for q in range(RPS // CHUNK):
        pltpu.sync_copy(rows_a, aggr_sh.at[pl.ds(s * RPS + q * CHUNK, CHUNK)])
    plsc.subcore_barrier()

    W = [[we_v[k, pl.ds(v * 16, 16)] for v in range(8)] for k in range(3)]
    B = [be_v[pl.ds(v * 16, 16)] for v in range(8)]

    def compute_chunk(jj, rows_v):
        for g in range(8):
            ea0 = ea_v[0, jj, pl.ds(g * 16, 16)]
            ea1 = ea_v[1, jj, pl.ds(g * 16, 16)]
            ea2 = ea_v[2, jj, pl.ds(g * 16, 16)]

            def l_body(l, _2, g=g, ea0=ea0, ea1=ea1, ea2=ea2):
                i = g * 16 + l
                s0 = _splat(ea0, l)
                s1 = _splat(ea1, l)
                s2 = _splat(ea2, l)
                for v in range(8):
                    sl = pl.ds(v * 16, 16)
                    hv = rows_v[i, sl]
                    m = hv + s0 * W[0][v] + s1 * W[1][v] + s2 * W[2][v] + B[v]
                    rows_v[i, sl] = jnp.maximum(m, 0.0)
                return 0

            lax.fori_loop(0, 16, l_body, 0)

    def gather(jj, rows_v, gsem):
        pltpu.async_copy(h_hbm.at[src_v.at[jj]], rows_v, gsem)

    def wait_gather(jj, rows_v, gsem):
        pltpu.make_async_copy(h_hbm.at[src_v.at[jj]], rows_v, gsem).wait()

    def scatter(jj, rows_v, ssem):
        pltpu.async_copy(rows_v, aggr_sh.at[dst_v.at[jj]], ssem, add=True)

    def wait_scatter(jj, rows_v, ssem):
        pltpu.make_async_copy(rows_v, aggr_sh.at[dst_v.at[jj]], ssem).wait()

    def sup_body(u, _):
        pltpu.sync_copy(src_hbm.at[t, pl.ds(u * SUP, SUP)], src_v)
        pltpu.sync_copy(dst_hbm.at[t, pl.ds(u * SUP, SUP)], dst_v)
        pltpu.sync_copy(ea_hbm.at[t, :, pl.ds(u * SUP, SUP)], ea_v)
        gather(0, rows_a, gsem_a)
        gather(1, rows_b, gsem_b)

        def pair_body(p, _2):
            jj0 = 2 * p
            jj1 = 2 * p + 1
            wait_gather(jj0, rows_a, gsem_a)
            compute_chunk(jj0, rows_a)
            scatter(jj0, rows_a, ssem_a)
            wait_gather(jj1, rows_b, gsem_b)
            compute_chunk(jj1, rows_b)
            scatter(jj1, rows_b, ssem_b)

            @pl.when(p < SUP // 2 - 1)
            def _refill():
                wait_scatter(jj0, rows_a, ssem_a)
                gather(jj0 + 2, rows_a, gsem_a)
                wait_scatter(jj1, rows_b, ssem_b)
                gather(jj1 + 2, rows_b, gsem_b)

            return 0

        lax.fori_loop(0, SUP // 2, pair_body, 0)
        wait_scatter(SUP - 2, rows_a, ssem_a)
        wait_scatter(SUP - 1, rows_b, ssem_b)
        return 0

    lax.fori_loop(0, NSUP, sup_body, 0)
    plsc.subcore_barrier()

    pltpu.sync_copy(aggr_sh.at[pl.ds(s * RPS, RPS)],
                    out_hbm.at[c, pl.ds(s * RPS, RPS)])


@functools.cache
def _get_edge_kernel():
  return functools.partial(
    pl.kernel,
    out_type=jax.ShapeDtypeStruct((NUM_CORES, N_PAD, H), jnp.float32),
    mesh=plsc.VectorSubcoreMesh(core_axis_name="c", subcore_axis_name="s",
                                num_cores=NUM_CORES, num_subcores=NUM_SUB),
    scratch_types=[
        pltpu.VMEM((SUP, CHUNK), jnp.int32),
        pltpu.VMEM((SUP, CHUNK), jnp.int32),
        pltpu.VMEM((3, SUP, CHUNK), jnp.float32),
        pltpu.VMEM((3, H), jnp.float32),
        pltpu.VMEM((H,), jnp.float32),
        pltpu.VMEM((CHUNK, H), jnp.float32),
        pltpu.VMEM((CHUNK, H), jnp.float32),
        pltpu.VMEM_SHARED((N_PAD, H), jnp.float32),
        pltpu.SemaphoreType.DMA,
        pltpu.SemaphoreType.DMA,
        pltpu.SemaphoreType.DMA,
        pltpu.SemaphoreType.DMA,
    ],
  )(_edge_body)


def _layer_body(h_ref, agg_ref, wa_ref, ba_ref, wb_ref, bb_ref, gm_ref,
                bt_ref, o_ref):
    z = h_ref[...] + agg_ref[0, :N, :] + agg_ref[1, :N, :]
    u = jnp.dot(z, wa_ref[...], preferred_element_type=jnp.float32)
    u = jnp.maximum(u + ba_ref[...], 0.0)
    y = jnp.dot(u, wb_ref[...], preferred_element_type=jnp.float32)
    y = y + bb_ref[...]
    mu = jnp.mean(y, axis=0, keepdims=True)
    yc = y - mu
    var = jnp.mean(yc * yc, axis=0, keepdims=True)
    o_ref[...] = jnp.maximum(
        gm_ref[...] * yc * lax.rsqrt(var + 1e-5) + bt_ref[...], 0.0)


_layer_tc = pl.pallas_call(
    _layer_body,
    out_shape=jax.ShapeDtypeStruct((N, H), jnp.float32),
)


def _final_body(h_ref, agg_ref, wa_ref, ba_ref, wb_ref, bb_ref, gm_ref,
                bt_ref, batch_ref, wf1_ref, bf1_ref, wf2_ref, bf2_ref, o_ref):
    z = h_ref[...] + agg_ref[0, :N, :] + agg_ref[1, :N, :]
    u = jnp.dot(z, wa_ref[...], preferred_element_type=jnp.float32)
    u = jnp.maximum(u + ba_ref[...], 0.0)
    y = jnp.dot(u, wb_ref[...], preferred_element_type=jnp.float32)
    y = y + bb_ref[...]
    mu = jnp.mean(y, axis=0, keepdims=True)
    yc = y - mu
    var = jnp.mean(yc * yc, axis=0, keepdims=True)
    h3 = jnp.maximum(
        gm_ref[...] * yc * lax.rsqrt(var + 1e-5) + bt_ref[...], 0.0)
    seg = lax.broadcasted_iota(jnp.int32, (NUM_GRAPHS, N), 0)
    onehot = (seg == jnp.broadcast_to(batch_ref[...], (NUM_GRAPHS, N))
              ).astype(jnp.float32)
    sums = jnp.dot(onehot, h3, preferred_element_type=jnp.float32)
    cnts = jnp.sum(onehot, axis=1, keepdims=True)
    pooled = sums / jnp.maximum(cnts, 1.0)
    t1 = jnp.dot(pooled, wf1_ref[...], preferred_element_type=jnp.float32)
    t1 = jnp.maximum(t1 + bf1_ref[...], 0.0)
    o_ref[...] = (jnp.dot(t1, wf2_ref[...], preferred_element_type=jnp.float32)
                  + bf2_ref[...])


_final_tc = pl.pallas_call(
    _final_body,
    out_shape=jax.ShapeDtypeStruct((NUM_GRAPHS, 1), jnp.float32),
)


def kernel(x, edge_index, edge_attr, batch,
           We1, be1, Wa1, ba1, Wb1, bb1, gamma1, beta1,
           We2, be2, Wa2, ba2, Wb2, bb2, gamma2, beta2,
           We3, be3, Wa3, ba3, Wb3, bb3, gamma3, beta3,
           Wf1, bf1, Wf2, bf2):
    pad = E_PAD - E
    src_p = jnp.concatenate(
        [edge_index[0], jnp.zeros((pad,), jnp.int32)]).reshape(
            NTILE, CPT, CHUNK)
    dummy = N + (jnp.arange(pad, dtype=jnp.int32) % (N_PAD - N))
    dst_p = jnp.concatenate([edge_index[1], dummy]).reshape(NTILE, CPT, CHUNK)
    ea_p = jnp.pad(edge_attr.T, ((0, 0), (0, pad))).reshape(
        3, NTILE, CPT, CHUNK).transpose(1, 0, 2, 3)
    zeros128 = jnp.zeros((CHUNK, H), jnp.float32)

    def edge(h, We, be):
        return _get_edge_kernel()(h, src_p, dst_p, ea_p, We, be, zeros128)

    r1 = lambda a: a.reshape(1, -1)
    agg1 = edge(x, We1, be1)
    h1 = _layer_tc(x, agg1, Wa1, r1(ba1), Wb1, r1(bb1), r1(gamma1), r1(beta1))
    agg2 = edge(h1, We2, be2)
    h2 = _layer_tc(h1, agg2, Wa2, r1(ba2), Wb2, r1(bb2), r1(gamma2), r1(beta2))
    agg3 = edge(h2, We3, be3)
    out = _final_tc(h2, agg3, Wa3, r1(ba3), Wb3, r1(bb3), r1(gamma3),
                    r1(beta3), batch.reshape(1, N), Wf1, r1(bf1),
                    Wf2, r1(bf2))
    return out
